# baseline (device time: 230346 ns/iter reference)
import jax
import jax.numpy as jnp
from jax import lax
from jax.experimental import pallas as pl
from jax.experimental.pallas import tpu as pltpu

N_DEV = 4
R, L = 0, 1


def kernel(A, B):
    M, _ = A.shape
    K, N = B.shape
    MB = M // N_DEV
    HB = MB // 2

    def body(a_ref, b_ref, out_ref, comm_ref, tmp_ref, send_sems,
             recv_sems, copy_sems):
        my = lax.axis_index("i")
        left = (my - 1) % N_DEV
        right = (my + 1) % N_DEV

        def row0(c, ring):
            return c * MB + ring * HB

        def partial(c, ring):
            return jnp.dot(
                a_ref[pl.ds(row0(c, ring), HB), :].astype(jnp.bfloat16),
                b_ref[...],
                preferred_element_type=jnp.float32,
            )

        def ring_copy(ring, send_slot, recv_slot, hop):
            return pltpu.make_async_remote_copy(
                src_ref=comm_ref.at[ring, send_slot],
                dst_ref=comm_ref.at[ring, recv_slot],
                send_sem=send_sems.at[ring, hop],
                recv_sem=recv_sems.at[ring, hop],
                device_id=(right if ring == R else left,),
                device_id_type=pl.DeviceIdType.MESH,
            )

        def accumulate(ring, slot):
            comm_ref[ring, slot, :, :] = (
                comm_ref[ring, slot, :, :].astype(jnp.float32)
                + tmp_ref[ring, :, :]
            ).astype(jnp.bfloat16)

        def store_from_tmp(ring, c):
            cp = pltpu.make_async_copy(
                tmp_ref.at[ring],
                out_ref.at[pl.ds(row0(c, ring), HB), :],
                copy_sems.at[ring],
            )
            cp.start()
            return cp

        barrier_sem = pltpu.get_barrier_semaphore()

        def neighbor_barrier():
            for nbr in (left, right):
                pl.semaphore_signal(
                    barrier_sem,
                    inc=1,
                    device_id=(nbr,),
                    device_id_type=pl.DeviceIdType.MESH,
                )
            pl.semaphore_wait(barrier_sem, 2)

        comm_ref[R, 0, :, :] = partial(my, R).astype(jnp.bfloat16)
        comm_ref[L, 0, :, :] = partial(my, L).astype(jnp.bfloat16)
        neighbor_barrier()
        ring_copy(R, 0, 1, 0).start()
        ring_copy(L, 0, 1, 0).start()
        tmp_ref[R, :, :] = partial((my - 1) % N_DEV, R)
        tmp_ref[L, :, :] = partial((my + 1) % N_DEV, L)
        for s in range(N_DEV - 1):
            send_slot = s % 2
            recv_slot = (s + 1) % 2
            rdma_r = ring_copy(R, send_slot, recv_slot, s)
            rdma_l = ring_copy(L, send_slot, recv_slot, s)
            rdma_r.wait()
            accumulate(R, recv_slot)
            if s < N_DEV - 2:
                ring_copy(R, recv_slot, send_slot, s + 1).start()
            rdma_l.wait()
            accumulate(L, recv_slot)
            if s < N_DEV - 2:
                ring_copy(L, recv_slot, send_slot, s + 1).start()
                tmp_ref[R, :, :] = partial((my - 2 - s) % N_DEV, R)
                tmp_ref[L, :, :] = partial((my + 2 + s) % N_DEV, L)

        own_r = (my + 1) % N_DEV
        own_l = (my - 1) % N_DEV
        tmp_ref[R, :, :] = comm_ref[R, 1, :, :].astype(jnp.float32)
        tmp_ref[L, :, :] = comm_ref[L, 1, :, :].astype(jnp.float32)
        cp_r = store_from_tmp(R, own_r)
        cp_l = store_from_tmp(L, own_l)
        cp_r.wait()
        cp_l.wait()

        neighbor_barrier()

        ring_copy(R, 1, 0, N_DEV - 1).start()
        ring_copy(L, 1, 0, N_DEV - 1).start()
        for h in range(N_DEV - 1):
            send_slot = (1 + h) % 2
            recv_slot = h % 2
            rdma_r = ring_copy(R, send_slot, recv_slot, N_DEV - 1 + h)
            rdma_l = ring_copy(L, send_slot, recv_slot, N_DEV - 1 + h)
            rdma_r.wait()
            if h < N_DEV - 2:
                ring_copy(R, recv_slot, send_slot, N_DEV + h).start()
            tmp_ref[R, :, :] = comm_ref[R, recv_slot, :, :].astype(jnp.float32)
            cp_r = store_from_tmp(R, (my - h) % N_DEV)
            rdma_l.wait()
            if h < N_DEV - 2:
                ring_copy(L, recv_slot, send_slot, N_DEV + h).start()
            tmp_ref[L, :, :] = comm_ref[L, recv_slot, :, :].astype(jnp.float32)
            cp_l = store_from_tmp(L, (my + h) % N_DEV)
            cp_r.wait()
            cp_l.wait()

    return pl.pallas_call(
        body,
        out_shape=jax.ShapeDtypeStruct((M, N), jnp.float32),
        in_specs=[
            pl.BlockSpec(memory_space=pltpu.VMEM),
            pl.BlockSpec(memory_space=pltpu.VMEM),
        ],
        out_specs=pl.BlockSpec(memory_space=pltpu.MemorySpace.HBM),
        scratch_shapes=[
            pltpu.VMEM((2, 2, HB, N), jnp.bfloat16),
            pltpu.VMEM((2, HB, N), jnp.float32),
            pltpu.SemaphoreType.DMA((2, 2 * (N_DEV - 1))),
            pltpu.SemaphoreType.DMA((2, 2 * (N_DEV - 1))),
            pltpu.SemaphoreType.DMA((2,)),
        ],
        compiler_params=pltpu.CompilerParams(
            collective_id=0, vmem_limit_bytes=100 * 1024 * 1024
        ),
    )(A, B.astype(jnp.bfloat16))


# device time: 208324 ns/iter; 1.1057x vs baseline; 1.1057x over previous
import jax
import jax.numpy as jnp
from jax import lax
from jax.experimental import pallas as pl
from jax.experimental.pallas import tpu as pltpu

N_DEV = 4
R, L = 0, 1
NS = 2


def kernel(A, B):
    M, _ = A.shape
    K, N = B.shape
    MB = M // N_DEV
    HB = MB // 2
    SC = HB // NS

    def body(a_ref, b_ref, out_ref, comm_ref, tmp_ref, send_sems, recv_sems,
             copy_sems):
        my = lax.axis_index("i")
        left = (my - 1) % N_DEV
        right = (my + 1) % N_DEV

        def row0(c, ring, j=0):
            return c * MB + ring * HB + j * SC

        def partial(c, ring, j=None):
            r0 = row0(c, ring) if j is None else row0(c, ring, j)
            rows = HB if j is None else SC
            return jnp.dot(
                a_ref[pl.ds(r0, rows), :],
                b_ref[...],
                preferred_element_type=jnp.float32,
            )

        def ring_copy(ring, send_slot, recv_slot, j, hop):
            return pltpu.make_async_remote_copy(
                src_ref=comm_ref.at[ring, send_slot, j],
                dst_ref=comm_ref.at[ring, recv_slot, j],
                send_sem=send_sems.at[ring, hop],
                recv_sem=recv_sems.at[ring, hop],
                device_id=(right if ring == R else left,),
                device_id_type=pl.DeviceIdType.MESH,
            )

        def accumulate(ring, slot, j):
            comm_ref[ring, slot, j, :, :] = (
                comm_ref[ring, slot, j, :, :].astype(jnp.float32)
                + tmp_ref[ring, pl.ds(j * SC, SC), :]
            ).astype(jnp.bfloat16)

        def store_sub(ring, c, j):
            cp = pltpu.make_async_copy(
                tmp_ref.at[ring, pl.ds(j * SC, SC), :],
                out_ref.at[pl.ds(row0(c, ring, j), SC), :],
                copy_sems.at[ring],
            )
            cp.start()
            return cp

        barrier_sem = pltpu.get_barrier_semaphore()

        def neighbor_barrier():
            for nbr in (left, right):
                pl.semaphore_signal(
                    barrier_sem,
                    inc=1,
                    device_id=(nbr,),
                    device_id_type=pl.DeviceIdType.MESH,
                )
            pl.semaphore_wait(barrier_sem, 2)

        neighbor_barrier()
        for j in range(NS):
            comm_ref[R, 0, j, :, :] = partial(my, R, j).astype(jnp.bfloat16)
            ring_copy(R, 0, 1, j, j).start()
            comm_ref[L, 0, j, :, :] = partial(my, L, j).astype(jnp.bfloat16)
            ring_copy(L, 0, 1, j, j).start()
        tmp_ref[R, :, :] = partial((my - 1) % N_DEV, R)
        tmp_ref[L, :, :] = partial((my + 1) % N_DEV, L)
        for s in range(N_DEV - 1):
            send_slot = s % 2
            recv_slot = (s + 1) % 2
            for j in range(NS):
                hop = 2 * s + j
                ring_copy(R, send_slot, recv_slot, j, hop).wait()
                accumulate(R, recv_slot, j)
                if s < N_DEV - 2:
                    ring_copy(R, recv_slot, send_slot, j, hop + 2).start()
                ring_copy(L, send_slot, recv_slot, j, hop).wait()
                accumulate(L, recv_slot, j)
                if s < N_DEV - 2:
                    ring_copy(L, recv_slot, send_slot, j, hop + 2).start()
            if s < N_DEV - 2:
                tmp_ref[R, :, :] = partial((my - 2 - s) % N_DEV, R)
                tmp_ref[L, :, :] = partial((my + 2 + s) % N_DEV, L)

        own_r = (my + 1) % N_DEV
        own_l = (my - 1) % N_DEV
        tmp_ref[R, :, :] = comm_ref[R, 1, :, :, :].reshape(HB, N).astype(
            jnp.float32
        )
        tmp_ref[L, :, :] = comm_ref[L, 1, :, :, :].reshape(HB, N).astype(
            jnp.float32
        )
        cps = [store_sub(ring, c, j)
               for ring, c in ((R, own_r), (L, own_l)) for j in range(NS)]
        for cp in cps:
            cp.wait()

        neighbor_barrier()

        for j in range(NS):
            ring_copy(R, 1, 0, j, 6 + j).start()
            ring_copy(L, 1, 0, j, 6 + j).start()
        for h in range(N_DEV - 1):
            send_slot = (1 + h) % 2
            recv_slot = h % 2
            cps = []
            for j in range(NS):
                hop = 6 + 2 * h + j
                ring_copy(R, send_slot, recv_slot, j, hop).wait()
                if h < N_DEV - 2:
                    ring_copy(R, recv_slot, send_slot, j, hop + 2).start()
                tmp_ref[R, pl.ds(j * SC, SC), :] = comm_ref[
                    R, recv_slot, j, :, :
                ].astype(jnp.float32)
                cps.append(store_sub(R, (my - h) % N_DEV, j))
                ring_copy(L, send_slot, recv_slot, j, hop).wait()
                if h < N_DEV - 2:
                    ring_copy(L, recv_slot, send_slot, j, hop + 2).start()
                tmp_ref[L, pl.ds(j * SC, SC), :] = comm_ref[
                    L, recv_slot, j, :, :
                ].astype(jnp.float32)
                cps.append(store_sub(L, (my + h) % N_DEV, j))
            for cp in cps:
                cp.wait()

    return pl.pallas_call(
        body,
        out_shape=jax.ShapeDtypeStruct((M, N), jnp.float32),
        in_specs=[
            pl.BlockSpec(memory_space=pltpu.VMEM),
            pl.BlockSpec(memory_space=pltpu.VMEM),
        ],
        out_specs=pl.BlockSpec(memory_space=pltpu.MemorySpace.HBM),
        scratch_shapes=[
            pltpu.VMEM((2, 2, NS, SC, N), jnp.bfloat16),
            pltpu.VMEM((2, HB, N), jnp.float32),
            pltpu.SemaphoreType.DMA((2, 4 * (N_DEV - 1))),
            pltpu.SemaphoreType.DMA((2, 4 * (N_DEV - 1))),
            pltpu.SemaphoreType.DMA((2,)),
        ],
        compiler_params=pltpu.CompilerParams(
            collective_id=0, vmem_limit_bytes=100 * 1024 * 1024
        ),
    )(A, B)


# device time: 203789 ns/iter; 1.1303x vs baseline; 1.0223x over previous
import jax
import jax.numpy as jnp
from jax import lax
from jax.experimental import pallas as pl
from jax.experimental.pallas import tpu as pltpu

N_DEV = 4
R, L = 0, 1
NS = 4


def kernel(A, B):
    M, _ = A.shape
    K, N = B.shape
    MB = M // N_DEV
    HB = MB // 2
    SC = HB // NS

    def body(a_ref, b_ref, out_ref, comm_ref, tmp_ref, send_sems, recv_sems,
             copy_sems):
        my = lax.axis_index("i")
        left = (my - 1) % N_DEV
        right = (my + 1) % N_DEV

        def row0(c, ring, j=0):
            return c * MB + ring * HB + j * SC

        def partial(c, ring, j=None):
            r0 = row0(c, ring) if j is None else row0(c, ring, j)
            rows = HB if j is None else SC
            return jnp.dot(
                a_ref[pl.ds(r0, rows), :],
                b_ref[...],
                preferred_element_type=jnp.float32,
            )

        def ring_copy(ring, send_slot, recv_slot, j, hop):
            return pltpu.make_async_remote_copy(
                src_ref=comm_ref.at[ring, send_slot, j],
                dst_ref=comm_ref.at[ring, recv_slot, j],
                send_sem=send_sems.at[ring, hop],
                recv_sem=recv_sems.at[ring, hop],
                device_id=(right if ring == R else left,),
                device_id_type=pl.DeviceIdType.MESH,
            )

        def accumulate(ring, slot, j):
            comm_ref[ring, slot, j, :, :] = (
                comm_ref[ring, slot, j, :, :].astype(jnp.float32)
                + tmp_ref[ring, pl.ds(j * SC, SC), :]
            ).astype(jnp.bfloat16)

        def store_sub(ring, c, j):
            cp = pltpu.make_async_copy(
                tmp_ref.at[ring, pl.ds(j * SC, SC), :],
                out_ref.at[pl.ds(row0(c, ring, j), SC), :],
                copy_sems.at[ring],
            )
            cp.start()
            return cp

        barrier_sem = pltpu.get_barrier_semaphore()

        def neighbor_barrier():
            for nbr in (left, right):
                pl.semaphore_signal(
                    barrier_sem,
                    inc=1,
                    device_id=(nbr,),
                    device_id_type=pl.DeviceIdType.MESH,
                )
            pl.semaphore_wait(barrier_sem, 2)

        neighbor_barrier()
        for j in range(NS):
            comm_ref[R, 0, j, :, :] = partial(my, R, j).astype(jnp.bfloat16)
            ring_copy(R, 0, 1, j, j).start()
            comm_ref[L, 0, j, :, :] = partial(my, L, j).astype(jnp.bfloat16)
            ring_copy(L, 0, 1, j, j).start()
        tmp_ref[R, :, :] = partial((my - 1) % N_DEV, R)
        tmp_ref[L, :, :] = partial((my + 1) % N_DEV, L)
        for s in range(N_DEV - 1):
            send_slot = s % 2
            recv_slot = (s + 1) % 2
            for j in range(NS):
                hop = NS * s + j
                ring_copy(R, send_slot, recv_slot, j, hop).wait()
                accumulate(R, recv_slot, j)
                if s < N_DEV - 2:
                    ring_copy(R, recv_slot, send_slot, j, hop + NS).start()
                ring_copy(L, send_slot, recv_slot, j, hop).wait()
                accumulate(L, recv_slot, j)
                if s < N_DEV - 2:
                    ring_copy(L, recv_slot, send_slot, j, hop + NS).start()
            if s < N_DEV - 2:
                tmp_ref[R, :, :] = partial((my - 2 - s) % N_DEV, R)
                tmp_ref[L, :, :] = partial((my + 2 + s) % N_DEV, L)

        neighbor_barrier()

        agb = NS * (N_DEV - 1)
        for j in range(NS):
            ring_copy(R, 1, 0, j, agb + j).start()
            ring_copy(L, 1, 0, j, agb + j).start()
        own_r = (my + 1) % N_DEV
        own_l = (my - 1) % N_DEV
        tmp_ref[R, :, :] = comm_ref[R, 1, :, :, :].reshape(HB, N).astype(
            jnp.float32
        )
        tmp_ref[L, :, :] = comm_ref[L, 1, :, :, :].reshape(HB, N).astype(
            jnp.float32
        )
        own_cps = [store_sub(ring, c, j)
                   for ring, c in ((R, own_r), (L, own_l)) for j in range(NS)]
        for h in range(N_DEV - 1):
            send_slot = (1 + h) % 2
            recv_slot = h % 2
            if h == 0:
                for cp in own_cps:
                    cp.wait()
            cps = []
            for j in range(NS):
                hop = agb + NS * h + j
                ring_copy(R, send_slot, recv_slot, j, hop).wait()
                if h < N_DEV - 2:
                    ring_copy(R, recv_slot, send_slot, j, hop + NS).start()
                tmp_ref[R, pl.ds(j * SC, SC), :] = comm_ref[
                    R, recv_slot, j, :, :
                ].astype(jnp.float32)
                cps.append(store_sub(R, (my - h) % N_DEV, j))
                ring_copy(L, send_slot, recv_slot, j, hop).wait()
                if h < N_DEV - 2:
                    ring_copy(L, recv_slot, send_slot, j, hop + NS).start()
                tmp_ref[L, pl.ds(j * SC, SC), :] = comm_ref[
                    L, recv_slot, j, :, :
                ].astype(jnp.float32)
                cps.append(store_sub(L, (my + h) % N_DEV, j))
            for cp in cps:
                cp.wait()

    return pl.pallas_call(
        body,
        out_shape=jax.ShapeDtypeStruct((M, N), jnp.float32),
        in_specs=[
            pl.BlockSpec(memory_space=pltpu.VMEM),
            pl.BlockSpec(memory_space=pltpu.VMEM),
        ],
        out_specs=pl.BlockSpec(memory_space=pltpu.MemorySpace.HBM),
        scratch_shapes=[
            pltpu.VMEM((2, 2, NS, SC, N), jnp.bfloat16),
            pltpu.VMEM((2, HB, N), jnp.float32),
            pltpu.SemaphoreType.DMA((2, 2 * NS * (N_DEV - 1))),
            pltpu.SemaphoreType.DMA((2, 2 * NS * (N_DEV - 1))),
            pltpu.SemaphoreType.DMA((2,)),
        ],
        compiler_params=pltpu.CompilerParams(
            collective_id=0, vmem_limit_bytes=100 * 1024 * 1024
        ),
    )(A, B)


# device time: 199698 ns/iter; 1.1535x vs baseline; 1.0205x over previous
import jax
import jax.numpy as jnp
from jax import lax
from jax.experimental import pallas as pl
from jax.experimental.pallas import tpu as pltpu

N_DEV = 4
R, L = 0, 1
NS = 4


def kernel(A, B):
    M, _ = A.shape
    K, N = B.shape
    MB = M // N_DEV
    HB = MB // 2
    SC = HB // NS

    def body(a_ref, b_ref, out_ref, comm_ref, tmp_ref, send_sems, recv_sems,
             copy_sems):
        my = lax.axis_index("i")
        left = (my - 1) % N_DEV
        right = (my + 1) % N_DEV

        def row0(c, ring, j=0):
            return c * MB + ring * HB + j * SC

        def partial(c, ring, j=None):
            r0 = row0(c, ring) if j is None else row0(c, ring, j)
            rows = HB if j is None else SC
            return jnp.dot(
                a_ref[pl.ds(r0, rows), :],
                b_ref[...],
                preferred_element_type=jnp.float32,
            )

        def ring_copy(ring, send_slot, recv_slot, j, hop):
            return pltpu.make_async_remote_copy(
                src_ref=comm_ref.at[ring, send_slot, j],
                dst_ref=comm_ref.at[ring, recv_slot, j],
                send_sem=send_sems.at[ring, hop],
                recv_sem=recv_sems.at[ring, hop],
                device_id=(right if ring == R else left,),
                device_id_type=pl.DeviceIdType.MESH,
            )

        def accumulate(ring, slot, j):
            comm_ref[ring, slot, j, :, :] = (
                comm_ref[ring, slot, j, :, :].astype(jnp.float32)
                + tmp_ref[ring, pl.ds(j * SC, SC), :]
            ).astype(jnp.bfloat16)

        def store_sub(ring, c, j):
            cp = pltpu.make_async_copy(
                tmp_ref.at[ring, pl.ds(j * SC, SC), :],
                out_ref.at[pl.ds(row0(c, ring, j), SC), :],
                copy_sems.at[ring],
            )
            cp.start()
            return cp

        barrier_sem = pltpu.get_barrier_semaphore()

        def neighbor_barrier():
            for nbr in (left, right):
                pl.semaphore_signal(
                    barrier_sem,
                    inc=1,
                    device_id=(nbr,),
                    device_id_type=pl.DeviceIdType.MESH,
                )
            pl.semaphore_wait(barrier_sem, 2)

        neighbor_barrier()
        for j in range(NS):
            comm_ref[R, 0, j, :, :] = partial(my, R, j).astype(jnp.bfloat16)
            ring_copy(R, 0, 1, j, j).start()
            comm_ref[L, 0, j, :, :] = partial(my, L, j).astype(jnp.bfloat16)
            ring_copy(L, 0, 1, j, j).start()
        tmp_ref[R, :, :] = partial((my - 1) % N_DEV, R)
        tmp_ref[L, :, :] = partial((my + 1) % N_DEV, L)
        agb = NS * (N_DEV - 1)
        for s in range(N_DEV - 1):
            send_slot = s % 2
            recv_slot = (s + 1) % 2
            for j in range(NS):
                hop = NS * s + j
                ring_copy(R, send_slot, recv_slot, j, hop).wait()
                accumulate(R, recv_slot, j)
                if s < N_DEV - 2:
                    ring_copy(R, recv_slot, send_slot, j, hop + NS).start()
                else:
                    ring_copy(R, 1, 0, j, agb + j).start()
                ring_copy(L, send_slot, recv_slot, j, hop).wait()
                accumulate(L, recv_slot, j)
                if s < N_DEV - 2:
                    ring_copy(L, recv_slot, send_slot, j, hop + NS).start()
                else:
                    ring_copy(L, 1, 0, j, agb + j).start()
            if s < N_DEV - 2:
                tmp_ref[R, :, :] = partial((my - 2 - s) % N_DEV, R)
                tmp_ref[L, :, :] = partial((my + 2 + s) % N_DEV, L)

        own_r = (my + 1) % N_DEV
        own_l = (my - 1) % N_DEV
        tmp_ref[R, :, :] = comm_ref[R, 1, :, :, :].reshape(HB, N).astype(
            jnp.float32
        )
        tmp_ref[L, :, :] = comm_ref[L, 1, :, :, :].reshape(HB, N).astype(
            jnp.float32
        )
        own_cps = [store_sub(ring, c, j)
                   for ring, c in ((R, own_r), (L, own_l)) for j in range(NS)]
        for h in range(N_DEV - 1):
            send_slot = (1 + h) % 2
            recv_slot = h % 2
            if h == 0:
                for cp in own_cps:
                    cp.wait()
            cps = []
            for j in range(NS):
                hop = agb + NS * h + j
                ring_copy(R, send_slot, recv_slot, j, hop).wait()
                if h < N_DEV - 2:
                    ring_copy(R, recv_slot, send_slot, j, hop + NS).start()
                tmp_ref[R, pl.ds(j * SC, SC), :] = comm_ref[
                    R, recv_slot, j, :, :
                ].astype(jnp.float32)
                cps.append(store_sub(R, (my - h) % N_DEV, j))
                ring_copy(L, send_slot, recv_slot, j, hop).wait()
                if h < N_DEV - 2:
                    ring_copy(L, recv_slot, send_slot, j, hop + NS).start()
                tmp_ref[L, pl.ds(j * SC, SC), :] = comm_ref[
                    L, recv_slot, j, :, :
                ].astype(jnp.float32)
                cps.append(store_sub(L, (my + h) % N_DEV, j))
            for cp in cps:
                cp.wait()

    return pl.pallas_call(
        body,
        out_shape=jax.ShapeDtypeStruct((M, N), jnp.float32),
        in_specs=[
            pl.BlockSpec(memory_space=pltpu.VMEM),
            pl.BlockSpec(memory_space=pltpu.VMEM),
        ],
        out_specs=pl.BlockSpec(memory_space=pltpu.MemorySpace.HBM),
        scratch_shapes=[
            pltpu.VMEM((2, 2, NS, SC, N), jnp.bfloat16),
            pltpu.VMEM((2, HB, N), jnp.float32),
            pltpu.SemaphoreType.DMA((2, 2 * NS * (N_DEV - 1))),
            pltpu.SemaphoreType.DMA((2, 2 * NS * (N_DEV - 1))),
            pltpu.SemaphoreType.DMA((2,)),
        ],
        compiler_params=pltpu.CompilerParams(
            collective_id=0, vmem_limit_bytes=100 * 1024 * 1024
        ),
    )(A, B)
